# baseline (speedup 1.0000x reference)
import functools

import jax
import jax.numpy as jnp
from jax import lax
from jax.experimental import pallas as pl
from jax.experimental.pallas import tpu as pltpu
from jax.experimental.pallas import tpu_sc as plsc

NC = 2
NS = 16
CH = 125


def _sc_mesh():
    return plsc.VectorSubcoreMesh(core_axis_name="c", subcore_axis_name="s")


def _tile_row_copy(s, n, src_at, dst_at):
    base = (n // NS) // 8 * 8
    rem = n - NS * base
    pltpu.sync_copy(src_at(s * base, base), dst_at(s * base, base))
    if rem:
        @pl.when(s == NS - 1)
        def _():
            pltpu.sync_copy(src_at(NS * base, rem), dst_at(NS * base, rem))


@functools.lru_cache(maxsize=None)
def _make_deg_kernel(n, e):
    nch = e // (NC * NS) // CH
    nblk = nch // 8

    @functools.partial(
        pl.kernel,
        mesh=_sc_mesh(),
        compiler_params=pltpu.CompilerParams(use_tc_tiling_on_sc=False),
        out_type=jax.ShapeDtypeStruct((NC, n, 16), jnp.float32),
        scratch_types=[
            pltpu.VMEM((nblk, 8, CH), jnp.int32),
            pltpu.VMEM((CH, 16), jnp.float32),
            pltpu.VMEM_SHARED((n, 16), jnp.float32),
            pltpu.SemaphoreType.DMA,
        ],
    )
    def deg_kernel(ei_hbm, ones_hbm, zero_hbm, out_hbm, dst_v, ones_v,
                   hist_sh, sem):
        c = lax.axis_index("c")
        s = lax.axis_index("s")
        blkbase = (c * NS + s) * nblk
        pltpu.sync_copy(ei_hbm.at[1, pl.ds(blkbase, nblk)], dst_v)
        pltpu.sync_copy(ones_hbm, ones_v)
        _tile_row_copy(s, n,
                       lambda o, l: zero_hbm.at[pl.ds(o, l)],
                       lambda o, l: hist_sh.at[pl.ds(o, l)])
        plsc.subcore_barrier()

        def body(j, carry):
            pltpu.async_copy(
                ones_v, hist_sh.at[dst_v.at[lax.div(j, 8), lax.rem(j, 8)]],
                sem, add=True)
            return carry

        lax.fori_loop(0, nch, body, 0)

        def drain(j, carry):
            pltpu.make_async_copy(
                ones_v, hist_sh.at[dst_v.at[lax.div(j, 8), lax.rem(j, 8)]],
                sem).wait()
            return carry

        lax.fori_loop(0, nch, drain, 0)
        plsc.subcore_barrier()
        _tile_row_copy(s, n,
                       lambda o, l: hist_sh.at[pl.ds(o, l)],
                       lambda o, l: out_hbm.at[c, pl.ds(o, l)])

    return deg_kernel


def _gather_scatter_stream(nblk, nb, ga, ei_hbm, blkbase, mp_hbm,
                           idx_v, rows_v, agg_sh, isem, gsem, ssem):
    ns = nb - ga
    nch = nblk * 8

    def idx_ref(k, comp):
        return idx_v.at[lax.div(k, 8), comp, lax.rem(k, 8)]

    def fire_blk(b):
        slot = b
        pltpu.async_copy(ei_hbm.at[0, blkbase + b], idx_v.at[slot, 0], isem)
        pltpu.async_copy(ei_hbm.at[1, blkbase + b], idx_v.at[slot, 1], isem)

    def wait_blk(b):
        slot = b
        for comp in range(2):
            pltpu.make_async_copy(ei_hbm.at[0, blkbase + b],
                                  idx_v.at[slot, comp], isem).wait()

    def fire_gather(k):
        pltpu.async_copy(mp_hbm.at[idx_ref(k, 0)],
                         rows_v.at[lax.rem(k, nb)], gsem)

    def wait_gather(k):
        pltpu.make_async_copy(mp_hbm.at[idx_ref(k, 0)],
                              rows_v.at[lax.rem(k, nb)], gsem).wait()

    def fire_scatter(k):
        pltpu.async_copy(rows_v.at[lax.rem(k, nb)],
                         agg_sh.at[idx_ref(k, 1)], ssem, add=True)

    def wait_scatter(k):
        pltpu.make_async_copy(rows_v.at[lax.rem(k, nb)],
                              agg_sh.at[idx_ref(k, 1)], ssem).wait()

    def stage(b, carry):
        fire_blk(b)
        return carry

    lax.fori_loop(0, nblk, stage, 0)

    def stage_wait(b, carry):
        wait_blk(b)
        return carry

    lax.fori_loop(0, nblk, stage_wait, 0)
    for k in range(ga):
        fire_gather(k)

    def outer(b, carry):
        for q in range(8):
            j = b * 8 + q

            @pl.when(j >= ns)
            def _():
                wait_scatter(j - ns)

            @pl.when(j + ga < nch)
            def _():
                fire_gather(j + ga)

            wait_gather(j)
            fire_scatter(j)
        return carry

    lax.fori_loop(0, nblk, outer, 0)
    for k in range(nch - ns, nch):
        wait_scatter(k)


@functools.lru_cache(maxsize=None)
def _make_agg_kernel(n, e, f, nb, ga):
    nblk = e // (NC * NS) // (8 * CH)

    @functools.partial(
        pl.kernel,
        mesh=_sc_mesh(),
        compiler_params=pltpu.CompilerParams(use_tc_tiling_on_sc=False),
        out_type=jax.ShapeDtypeStruct((NC, n, f), jnp.float32),
        scratch_types=[
            pltpu.VMEM((nblk, 2, 8, CH), jnp.int32),
            pltpu.VMEM((nb, CH, f), jnp.float32),
            pltpu.VMEM_SHARED((n, f), jnp.float32),
            pltpu.SemaphoreType.DMA,
            pltpu.SemaphoreType.DMA,
            pltpu.SemaphoreType.DMA,
        ],
    )
    def agg_kernel(ei_hbm, mp_hbm, zero_hbm, out_hbm,
                   idx_v, rows_v, agg_sh, isem, gsem, ssem):
        c = lax.axis_index("c")
        s = lax.axis_index("s")
        blkbase = (c * NS + s) * nblk
        _tile_row_copy(s, n,
                       lambda o, l: zero_hbm.at[pl.ds(o, l)],
                       lambda o, l: agg_sh.at[pl.ds(o, l)])
        plsc.subcore_barrier()
        _gather_scatter_stream(nblk, nb, ga, ei_hbm, blkbase, mp_hbm,
                               idx_v, rows_v, agg_sh, isem, gsem, ssem)
        plsc.subcore_barrier()
        _tile_row_copy(s, n,
                       lambda o, l: agg_sh.at[pl.ds(o, l)],
                       lambda o, l: out_hbm.at[c, pl.ds(o, l)])

    return agg_kernel


@functools.lru_cache(maxsize=None)
def _make_agg_split_kernel(n, e, half, nb, ga):
    nblk = e // NS // (8 * CH)

    @functools.partial(
        pl.kernel,
        mesh=_sc_mesh(),
        compiler_params=pltpu.CompilerParams(use_tc_tiling_on_sc=False),
        out_type=jax.ShapeDtypeStruct((NC * n, half), jnp.float32),
        scratch_types=[
            pltpu.VMEM((nblk, 2, 8, CH), jnp.int32),
            pltpu.VMEM((nb, CH, half), jnp.float32),
            pltpu.VMEM_SHARED((n, half), jnp.float32),
            pltpu.SemaphoreType.DMA,
            pltpu.SemaphoreType.DMA,
            pltpu.SemaphoreType.DMA,
        ],
    )
    def agg_kernel(ei_hbm, mplo_hbm, mphi_hbm, zero_hbm, out_hbm,
                   idx_v, rows_v, agg_sh, isem, gsem, ssem):
        c = lax.axis_index("c")
        s = lax.axis_index("s")
        blkbase = s * nblk
        _tile_row_copy(s, n,
                       lambda o, l: zero_hbm.at[pl.ds(o, l)],
                       lambda o, l: agg_sh.at[pl.ds(o, l)])
        plsc.subcore_barrier()

        @pl.when(c == 0)
        def _():
            _gather_scatter_stream(nblk, nb, ga, ei_hbm, blkbase, mplo_hbm,
                                   idx_v, rows_v, agg_sh, isem, gsem, ssem)

        @pl.when(c == 1)
        def _():
            _gather_scatter_stream(nblk, nb, ga, ei_hbm, blkbase, mphi_hbm,
                                   idx_v, rows_v, agg_sh, isem, gsem, ssem)

        plsc.subcore_barrier()
        _tile_row_copy(s, n,
                       lambda o, l: agg_sh.at[pl.ds(o, l)],
                       lambda o, l: out_hbm.at[pl.ds(c * n + o, l)])

    return agg_kernel


def _prep_body(degh_ref, x_ref, w_ref, dinv_ref, mplo_ref, mphi_ref):
    half = mplo_ref.shape[1]
    deg = degh_ref[0, :, 0] + degh_ref[1, :, 0] + 1.0
    dinv = lax.rsqrt(deg)[:, None]
    dinv_ref[...] = dinv
    m = dinv * jnp.dot(x_ref[...], w_ref[...])
    mplo_ref[...] = m[:, :half]
    mphi_ref[...] = m[:, half:]


def _mid_split_body(slo_ref, shi_ref, mplo_ref, mphi_ref, dinv_ref, b_ref,
                    w_ref, mn_ref):
    dinv = dinv_ref[...]
    agg = jnp.concatenate([slo_ref[...] + mplo_ref[...],
                           shi_ref[...] + mphi_ref[...]], axis=-1)
    h = jnp.tanh(dinv * agg + b_ref[...])
    mn_ref[...] = dinv * jnp.dot(h, w_ref[...])


def _mid_body(s_ref, mp_ref, dinv_ref, b_ref, w_ref, mn_ref):
    dinv = dinv_ref[...]
    h = jnp.tanh(dinv * (s_ref[0] + s_ref[1] + mp_ref[...]) + b_ref[...])
    mn_ref[...] = dinv * jnp.dot(h, w_ref[...])


def _final_body(s_ref, mp_ref, dinv_ref, b_ref, wc_ref, bc_ref,
                out_ref, h_ref):
    h = jnp.tanh(dinv_ref[...] * (s_ref[0] + s_ref[1] + mp_ref[...])
                 + b_ref[...])
    h_ref[...] = h
    out_ref[...] = jnp.dot(h, wc_ref[...]) + bc_ref[...]


def kernel(x, edge_index, W1, b1, W2, b2, W3, b3, Wc, bc):
    n, d_in = x.shape
    e = edge_index.shape[1]
    f1, f2, f3 = W1.shape[1], W2.shape[1], W3.shape[1]
    half = f1 // 2
    ncls = Wc.shape[1]
    br = 2000
    grid = (n // br,)
    nb = n // br

    ei4 = edge_index.reshape(2, e // (8 * CH), 8, CH)
    ones16 = jnp.ones((CH, 16), jnp.float32)
    zeros = {f: jnp.zeros((n, f), jnp.float32) for f in {16, half, f2, f3}}

    degh = _make_deg_kernel(n, e)(ei4, ones16, zeros[16])

    rows = lambda i: (i, 0)
    rows_hi = lambda i: (i + nb, 0)
    fixed = lambda i: (0, 0)
    part = lambda i: (0, i, 0)

    dinv, mplo, mphi = pl.pallas_call(
        _prep_body,
        grid=grid,
        in_specs=[
            pl.BlockSpec((NC, br, 16), part),
            pl.BlockSpec((br, d_in), rows),
            pl.BlockSpec((d_in, f1), fixed),
        ],
        out_specs=[
            pl.BlockSpec((br, 1), rows),
            pl.BlockSpec((br, half), rows),
            pl.BlockSpec((br, half), rows),
        ],
        out_shape=[
            jax.ShapeDtypeStruct((n, 1), jnp.float32),
            jax.ShapeDtypeStruct((n, half), jnp.float32),
            jax.ShapeDtypeStruct((n, half), jnp.float32),
        ],
    )(degh, x, W1)

    s1f = _make_agg_split_kernel(n, e, half, 6, 3)(
        ei4, mplo, mphi, zeros[half])

    mp2 = pl.pallas_call(
        _mid_split_body,
        grid=grid,
        in_specs=[
            pl.BlockSpec((br, half), rows),
            pl.BlockSpec((br, half), rows_hi),
            pl.BlockSpec((br, half), rows),
            pl.BlockSpec((br, half), rows),
            pl.BlockSpec((br, 1), rows),
            pl.BlockSpec((1, f1), fixed),
            pl.BlockSpec((f1, f2), fixed),
        ],
        out_specs=pl.BlockSpec((br, f2), rows),
        out_shape=jax.ShapeDtypeStruct((n, f2), jnp.float32),
    )(s1f, s1f, mplo, mphi, dinv, b1.reshape(1, f1), W2)

    s2 = _make_agg_kernel(n, e, f2, 8, 4)(ei4, mp2, zeros[f2])

    mp3 = pl.pallas_call(
        _mid_body,
        grid=grid,
        in_specs=[
            pl.BlockSpec((NC, br, f2), part),
            pl.BlockSpec((br, f2), rows),
            pl.BlockSpec((br, 1), rows),
            pl.BlockSpec((1, f2), fixed),
            pl.BlockSpec((f2, f3), fixed),
        ],
        out_specs=pl.BlockSpec((br, f3), rows),
        out_shape=jax.ShapeDtypeStruct((n, f3), jnp.float32),
    )(s2, mp2, dinv, b2.reshape(1, f2), W3)

    s3 = _make_agg_kernel(n, e, f3, 16, 8)(ei4, mp3, zeros[f3])

    out, h3 = pl.pallas_call(
        _final_body,
        grid=grid,
        in_specs=[
            pl.BlockSpec((NC, br, f3), part),
            pl.BlockSpec((br, f3), rows),
            pl.BlockSpec((br, 1), rows),
            pl.BlockSpec((1, f3), fixed),
            pl.BlockSpec((f3, ncls), fixed),
            pl.BlockSpec((1, ncls), fixed),
        ],
        out_specs=[
            pl.BlockSpec((br, ncls), rows),
            pl.BlockSpec((br, f3), rows),
        ],
        out_shape=[
            jax.ShapeDtypeStruct((n, ncls), jnp.float32),
            jax.ShapeDtypeStruct((n, f3), jnp.float32),
        ],
    )(s3, mp3, dinv, b3.reshape(1, f3), Wc, bc.reshape(1, ncls))

    return (out, h3)


# speedup vs baseline: 1.0062x; 1.0062x over previous
import functools

import jax
import jax.numpy as jnp
from jax import lax
from jax.experimental import pallas as pl
from jax.experimental.pallas import tpu as pltpu
from jax.experimental.pallas import tpu_sc as plsc

NC = 2
NS = 16
CH = 125


def _sc_mesh():
    return plsc.VectorSubcoreMesh(core_axis_name="c", subcore_axis_name="s")


def _tile_row_copy(s, n, src_at, dst_at):
    base = (n // NS) // 8 * 8
    rem = n - NS * base
    pltpu.sync_copy(src_at(s * base, base), dst_at(s * base, base))
    if rem:
        @pl.when(s == NS - 1)
        def _():
            pltpu.sync_copy(src_at(NS * base, rem), dst_at(NS * base, rem))


@functools.lru_cache(maxsize=None)
def _make_deg_kernel(n, e):
    nch = e // (NC * NS) // CH
    nblk = nch // 8

    @functools.partial(
        pl.kernel,
        mesh=_sc_mesh(),
        compiler_params=pltpu.CompilerParams(use_tc_tiling_on_sc=False),
        out_type=jax.ShapeDtypeStruct((NC, n, 16), jnp.float32),
        scratch_types=[
            pltpu.VMEM((nblk, 8, CH), jnp.int32),
            pltpu.VMEM((CH, 16), jnp.float32),
            pltpu.VMEM_SHARED((n, 16), jnp.float32),
            pltpu.SemaphoreType.DMA,
        ],
    )
    def deg_kernel(ei_hbm, ones_hbm, zero_hbm, out_hbm, dst_v, ones_v,
                   hist_sh, sem):
        c = lax.axis_index("c")
        s = lax.axis_index("s")
        blkbase = (c * NS + s) * nblk
        pltpu.sync_copy(ei_hbm.at[1, pl.ds(blkbase, nblk)], dst_v)
        pltpu.sync_copy(ones_hbm, ones_v)
        _tile_row_copy(s, n,
                       lambda o, l: zero_hbm.at[pl.ds(o, l)],
                       lambda o, l: hist_sh.at[pl.ds(o, l)])
        plsc.subcore_barrier()

        def body(j, carry):
            pltpu.async_copy(
                ones_v, hist_sh.at[dst_v.at[lax.div(j, 8), lax.rem(j, 8)]],
                sem, add=True)
            return carry

        lax.fori_loop(0, nch, body, 0)

        def drain(j, carry):
            pltpu.make_async_copy(
                ones_v, hist_sh.at[dst_v.at[lax.div(j, 8), lax.rem(j, 8)]],
                sem).wait()
            return carry

        lax.fori_loop(0, nch, drain, 0)
        plsc.subcore_barrier()
        _tile_row_copy(s, n,
                       lambda o, l: hist_sh.at[pl.ds(o, l)],
                       lambda o, l: out_hbm.at[c, pl.ds(o, l)])

    return deg_kernel


def _gather_scatter_stream(nblk, nb, ga, ei_hbm, blkbase, mp_hbm,
                           idx_v, rows_v, agg_sh, isem0, isem1, gsem, ssem):
    ns = nb - ga
    nch = nblk * 8
    isems = (isem0, isem1)

    def idx_ref(k, comp):
        return idx_v.at[lax.rem(lax.div(k, 8), 4), comp, lax.rem(k, 8)]

    def fire_blk(b, sem):
        slot = lax.rem(b, 4)
        pltpu.async_copy(ei_hbm.at[0, blkbase + b], idx_v.at[slot, 0], sem)
        pltpu.async_copy(ei_hbm.at[1, blkbase + b], idx_v.at[slot, 1], sem)

    def wait_blk(b, sem):
        slot = lax.rem(b, 4)
        for comp in range(2):
            pltpu.make_async_copy(ei_hbm.at[0, blkbase + b],
                                  idx_v.at[slot, comp], sem).wait()

    def fire_gather(k):
        pltpu.async_copy(mp_hbm.at[idx_ref(k, 0)],
                         rows_v.at[lax.rem(k, nb)], gsem)

    def wait_gather(k):
        pltpu.make_async_copy(mp_hbm.at[idx_ref(k, 0)],
                              rows_v.at[lax.rem(k, nb)], gsem).wait()

    def fire_scatter(k):
        pltpu.async_copy(rows_v.at[lax.rem(k, nb)],
                         agg_sh.at[idx_ref(k, 1)], ssem, add=True)

    def wait_scatter(k):
        pltpu.make_async_copy(rows_v.at[lax.rem(k, nb)],
                              agg_sh.at[idx_ref(k, 1)], ssem).wait()

    fire_blk(0, isems[0])
    fire_blk(1, isems[1])
    wait_blk(0, isems[0])
    for k in range(ga):
        fire_gather(k)

    def outer(p, carry):
        for pb in range(2):
            b = 2 * p + pb

            @pl.when(b + 2 < nblk)
            def _():
                fire_blk(b + 2, isems[pb])

            @pl.when(b + 1 < nblk)
            def _():
                wait_blk(b + 1, isems[1 - pb])

            for q in range(8):
                j = b * 8 + q

                @pl.when(j >= ns)
                def _():
                    wait_scatter(j - ns)

                @pl.when(j + ga < nch)
                def _():
                    fire_gather(j + ga)

                wait_gather(j)
                fire_scatter(j)
        return carry

    lax.fori_loop(0, nblk // 2, outer, 0)
    for k in range(nch - ns, nch):
        wait_scatter(k)


@functools.lru_cache(maxsize=None)
def _make_agg_kernel(n, e, f, nb, ga):
    nblk = e // (NC * NS) // (8 * CH)

    @functools.partial(
        pl.kernel,
        mesh=_sc_mesh(),
        compiler_params=pltpu.CompilerParams(use_tc_tiling_on_sc=False),
        out_type=jax.ShapeDtypeStruct((NC, n, f), jnp.float32),
        scratch_types=[
            pltpu.VMEM((4, 2, 8, CH), jnp.int32),
            pltpu.VMEM((nb, CH, f), jnp.float32),
            pltpu.VMEM_SHARED((n, f), jnp.float32),
            pltpu.SemaphoreType.DMA,
            pltpu.SemaphoreType.DMA,
            pltpu.SemaphoreType.DMA,
            pltpu.SemaphoreType.DMA,
        ],
    )
    def agg_kernel(ei_hbm, mp_hbm, zero_hbm, out_hbm,
                   idx_v, rows_v, agg_sh, isem0, isem1, gsem, ssem):
        c = lax.axis_index("c")
        s = lax.axis_index("s")
        blkbase = (c * NS + s) * nblk
        _tile_row_copy(s, n,
                       lambda o, l: zero_hbm.at[pl.ds(o, l)],
                       lambda o, l: agg_sh.at[pl.ds(o, l)])
        plsc.subcore_barrier()
        _gather_scatter_stream(nblk, nb, ga, ei_hbm, blkbase, mp_hbm,
                               idx_v, rows_v, agg_sh, isem0, isem1,
                               gsem, ssem)
        plsc.subcore_barrier()
        _tile_row_copy(s, n,
                       lambda o, l: agg_sh.at[pl.ds(o, l)],
                       lambda o, l: out_hbm.at[c, pl.ds(o, l)])

    return agg_kernel


@functools.lru_cache(maxsize=None)
def _make_agg_split_kernel(n, e, half, nb, ga):
    nblk = e // NS // (8 * CH)

    @functools.partial(
        pl.kernel,
        mesh=_sc_mesh(),
        compiler_params=pltpu.CompilerParams(use_tc_tiling_on_sc=False),
        out_type=jax.ShapeDtypeStruct((NC * n, half), jnp.float32),
        scratch_types=[
            pltpu.VMEM((4, 2, 8, CH), jnp.int32),
            pltpu.VMEM((nb, CH, half), jnp.float32),
            pltpu.VMEM_SHARED((n, half), jnp.float32),
            pltpu.SemaphoreType.DMA,
            pltpu.SemaphoreType.DMA,
            pltpu.SemaphoreType.DMA,
            pltpu.SemaphoreType.DMA,
        ],
    )
    def agg_kernel(ei_hbm, mplo_hbm, mphi_hbm, zero_hbm, out_hbm,
                   idx_v, rows_v, agg_sh, isem0, isem1, gsem, ssem):
        c = lax.axis_index("c")
        s = lax.axis_index("s")
        blkbase = s * nblk
        _tile_row_copy(s, n,
                       lambda o, l: zero_hbm.at[pl.ds(o, l)],
                       lambda o, l: agg_sh.at[pl.ds(o, l)])
        plsc.subcore_barrier()

        @pl.when(c == 0)
        def _():
            _gather_scatter_stream(nblk, nb, ga, ei_hbm, blkbase, mplo_hbm,
                                   idx_v, rows_v, agg_sh, isem0, isem1,
                                   gsem, ssem)

        @pl.when(c == 1)
        def _():
            _gather_scatter_stream(nblk, nb, ga, ei_hbm, blkbase, mphi_hbm,
                                   idx_v, rows_v, agg_sh, isem0, isem1,
                                   gsem, ssem)

        plsc.subcore_barrier()
        _tile_row_copy(s, n,
                       lambda o, l: agg_sh.at[pl.ds(o, l)],
                       lambda o, l: out_hbm.at[pl.ds(c * n + o, l)])

    return agg_kernel


def _prep_body(degh_ref, x_ref, w_ref, dinv_ref, mplo_ref, mphi_ref):
    half = mplo_ref.shape[1]
    deg = degh_ref[0, :, 0] + degh_ref[1, :, 0] + 1.0
    dinv = lax.rsqrt(deg)[:, None]
    dinv_ref[...] = dinv
    m = dinv * jnp.dot(x_ref[...], w_ref[...])
    mplo_ref[...] = m[:, :half]
    mphi_ref[...] = m[:, half:]


def _mid_split_body(slo_ref, shi_ref, mplo_ref, mphi_ref, dinv_ref, b_ref,
                    w_ref, mn_ref):
    dinv = dinv_ref[...]
    agg = jnp.concatenate([slo_ref[...] + mplo_ref[...],
                           shi_ref[...] + mphi_ref[...]], axis=-1)
    h = jnp.tanh(dinv * agg + b_ref[...])
    mn_ref[...] = dinv * jnp.dot(h, w_ref[...])


def _mid_body(s_ref, mp_ref, dinv_ref, b_ref, w_ref, mn_ref):
    dinv = dinv_ref[...]
    h = jnp.tanh(dinv * (s_ref[0] + s_ref[1] + mp_ref[...]) + b_ref[...])
    mn_ref[...] = dinv * jnp.dot(h, w_ref[...])


def _final_body(s_ref, mp_ref, dinv_ref, b_ref, wc_ref, bc_ref,
                out_ref, h_ref):
    h = jnp.tanh(dinv_ref[...] * (s_ref[0] + s_ref[1] + mp_ref[...])
                 + b_ref[...])
    h_ref[...] = h
    out_ref[...] = jnp.dot(h, wc_ref[...]) + bc_ref[...]


def kernel(x, edge_index, W1, b1, W2, b2, W3, b3, Wc, bc):
    n, d_in = x.shape
    e = edge_index.shape[1]
    f1, f2, f3 = W1.shape[1], W2.shape[1], W3.shape[1]
    half = f1 // 2
    ncls = Wc.shape[1]
    br = 2000
    grid = (n // br,)
    nb = n // br

    ei4 = edge_index.reshape(2, e // (8 * CH), 8, CH)
    ones16 = jnp.ones((CH, 16), jnp.float32)
    zeros = {f: jnp.zeros((n, f), jnp.float32) for f in {16, half, f2, f3}}

    degh = _make_deg_kernel(n, e)(ei4, ones16, zeros[16])

    rows = lambda i: (i, 0)
    rows_hi = lambda i: (i + nb, 0)
    fixed = lambda i: (0, 0)
    part = lambda i: (0, i, 0)

    dinv, mplo, mphi = pl.pallas_call(
        _prep_body,
        grid=grid,
        in_specs=[
            pl.BlockSpec((NC, br, 16), part),
            pl.BlockSpec((br, d_in), rows),
            pl.BlockSpec((d_in, f1), fixed),
        ],
        out_specs=[
            pl.BlockSpec((br, 1), rows),
            pl.BlockSpec((br, half), rows),
            pl.BlockSpec((br, half), rows),
        ],
        out_shape=[
            jax.ShapeDtypeStruct((n, 1), jnp.float32),
            jax.ShapeDtypeStruct((n, half), jnp.float32),
            jax.ShapeDtypeStruct((n, half), jnp.float32),
        ],
    )(degh, x, W1)

    s1f = _make_agg_split_kernel(n, e, half, 9, 5)(
        ei4, mplo, mphi, zeros[half])

    mp2 = pl.pallas_call(
        _mid_split_body,
        grid=grid,
        in_specs=[
            pl.BlockSpec((br, half), rows),
            pl.BlockSpec((br, half), rows_hi),
            pl.BlockSpec((br, half), rows),
            pl.BlockSpec((br, half), rows),
            pl.BlockSpec((br, 1), rows),
            pl.BlockSpec((1, f1), fixed),
            pl.BlockSpec((f1, f2), fixed),
        ],
        out_specs=pl.BlockSpec((br, f2), rows),
        out_shape=jax.ShapeDtypeStruct((n, f2), jnp.float32),
    )(s1f, s1f, mplo, mphi, dinv, b1.reshape(1, f1), W2)

    s2 = _make_agg_kernel(n, e, f2, 9, 5)(ei4, mp2, zeros[f2])

    mp3 = pl.pallas_call(
        _mid_body,
        grid=grid,
        in_specs=[
            pl.BlockSpec((NC, br, f2), part),
            pl.BlockSpec((br, f2), rows),
            pl.BlockSpec((br, 1), rows),
            pl.BlockSpec((1, f2), fixed),
            pl.BlockSpec((f2, f3), fixed),
        ],
        out_specs=pl.BlockSpec((br, f3), rows),
        out_shape=jax.ShapeDtypeStruct((n, f3), jnp.float32),
    )(s2, mp2, dinv, b2.reshape(1, f2), W3)

    s3 = _make_agg_kernel(n, e, f3, 16, 8)(ei4, mp3, zeros[f3])

    out, h3 = pl.pallas_call(
        _final_body,
        grid=grid,
        in_specs=[
            pl.BlockSpec((NC, br, f3), part),
            pl.BlockSpec((br, f3), rows),
            pl.BlockSpec((br, 1), rows),
            pl.BlockSpec((1, f3), fixed),
            pl.BlockSpec((f3, ncls), fixed),
            pl.BlockSpec((1, ncls), fixed),
        ],
        out_specs=[
            pl.BlockSpec((br, ncls), rows),
            pl.BlockSpec((br, f3), rows),
        ],
        out_shape=[
            jax.ShapeDtypeStruct((n, ncls), jnp.float32),
            jax.ShapeDtypeStruct((n, f3), jnp.float32),
        ],
    )(s3, mp3, dinv, b3.reshape(1, f3), Wc, bc.reshape(1, ncls))

    return (out, h3)


# speedup vs baseline: 1.0093x; 1.0030x over previous
import functools

import jax
import jax.numpy as jnp
from jax import lax
from jax.experimental import pallas as pl
from jax.experimental.pallas import tpu as pltpu
from jax.experimental.pallas import tpu_sc as plsc

NC = 2
NS = 16
CH = 125


def _sc_mesh():
    return plsc.VectorSubcoreMesh(core_axis_name="c", subcore_axis_name="s")


def _tile_row_copy(s, n, src_at, dst_at):
    base = (n // NS) // 8 * 8
    rem = n - NS * base
    pltpu.sync_copy(src_at(s * base, base), dst_at(s * base, base))
    if rem:
        @pl.when(s == NS - 1)
        def _():
            pltpu.sync_copy(src_at(NS * base, rem), dst_at(NS * base, rem))


@functools.lru_cache(maxsize=None)
def _make_deg_kernel(n, e):
    nch = e // (NC * NS) // CH
    nblk = nch // 8

    @functools.partial(
        pl.kernel,
        mesh=_sc_mesh(),
        compiler_params=pltpu.CompilerParams(use_tc_tiling_on_sc=False),
        out_type=jax.ShapeDtypeStruct((NC, n, 16), jnp.float32),
        scratch_types=[
            pltpu.VMEM((nblk, 8, CH), jnp.int32),
            pltpu.VMEM((CH, 16), jnp.float32),
            pltpu.VMEM_SHARED((n, 16), jnp.float32),
            pltpu.SemaphoreType.DMA,
        ],
    )
    def deg_kernel(ei_hbm, ones_hbm, zero_hbm, out_hbm, dst_v, ones_v,
                   hist_sh, sem):
        c = lax.axis_index("c")
        s = lax.axis_index("s")
        blkbase = (c * NS + s) * nblk
        pltpu.sync_copy(ei_hbm.at[1, pl.ds(blkbase, nblk)], dst_v)
        pltpu.sync_copy(ones_hbm, ones_v)
        _tile_row_copy(s, n,
                       lambda o, l: zero_hbm.at[pl.ds(o, l)],
                       lambda o, l: hist_sh.at[pl.ds(o, l)])
        plsc.subcore_barrier()

        def body(j, carry):
            pltpu.async_copy(
                ones_v, hist_sh.at[dst_v.at[lax.div(j, 8), lax.rem(j, 8)]],
                sem, add=True)
            return carry

        lax.fori_loop(0, nch, body, 0)

        def drain(j, carry):
            pltpu.make_async_copy(
                ones_v, hist_sh.at[dst_v.at[lax.div(j, 8), lax.rem(j, 8)]],
                sem).wait()
            return carry

        lax.fori_loop(0, nch, drain, 0)
        plsc.subcore_barrier()
        _tile_row_copy(s, n,
                       lambda o, l: hist_sh.at[pl.ds(o, l)],
                       lambda o, l: out_hbm.at[c, pl.ds(o, l)])

    return deg_kernel


def _gather_scatter_stream(nblk, nb, ga, ei_hbm, blkbase, mp_hbm,
                           idx_v, rows_v, agg_sh, isem0, isem1, gsem, ssem):
    ns = nb - ga
    nch = nblk * 8
    isems = (isem0, isem1)

    def idx_ref(k, comp):
        return idx_v.at[lax.rem(lax.div(k, 8), 4), comp, lax.rem(k, 8)]

    def fire_blk(b, sem):
        slot = lax.rem(b, 4)
        pltpu.async_copy(ei_hbm.at[0, blkbase + b], idx_v.at[slot, 0], sem)
        pltpu.async_copy(ei_hbm.at[1, blkbase + b], idx_v.at[slot, 1], sem)

    def wait_blk(b, sem):
        slot = lax.rem(b, 4)
        for comp in range(2):
            pltpu.make_async_copy(ei_hbm.at[0, blkbase + b],
                                  idx_v.at[slot, comp], sem).wait()

    def fire_gather(k):
        pltpu.async_copy(mp_hbm.at[idx_ref(k, 0)],
                         rows_v.at[lax.rem(k, nb)], gsem)

    def wait_gather(k):
        pltpu.make_async_copy(mp_hbm.at[idx_ref(k, 0)],
                              rows_v.at[lax.rem(k, nb)], gsem).wait()

    def fire_scatter(k):
        pltpu.async_copy(rows_v.at[lax.rem(k, nb)],
                         agg_sh.at[idx_ref(k, 1)], ssem, add=True)

    def wait_scatter(k):
        pltpu.make_async_copy(rows_v.at[lax.rem(k, nb)],
                              agg_sh.at[idx_ref(k, 1)], ssem).wait()

    fire_blk(0, isems[0])
    fire_blk(1, isems[1])
    wait_blk(0, isems[0])
    for k in range(ga):
        fire_gather(k)

    def outer(p, carry):
        for pb in range(2):
            b = 2 * p + pb

            @pl.when(b + 2 < nblk)
            def _():
                fire_blk(b + 2, isems[pb])

            @pl.when(b + 1 < nblk)
            def _():
                wait_blk(b + 1, isems[1 - pb])

            for q in range(8):
                j = b * 8 + q

                @pl.when(j >= ns)
                def _():
                    wait_scatter(j - ns)

                @pl.when(j + ga < nch)
                def _():
                    fire_gather(j + ga)

                wait_gather(j)
                fire_scatter(j)
        return carry

    lax.fori_loop(0, nblk // 2, outer, 0)
    for k in range(nch - ns, nch):
        wait_scatter(k)


@functools.lru_cache(maxsize=None)
def _make_agg_kernel(n, e, f, nb, ga):
    nblk = e // (NC * NS) // (8 * CH)

    @functools.partial(
        pl.kernel,
        mesh=_sc_mesh(),
        compiler_params=pltpu.CompilerParams(use_tc_tiling_on_sc=False),
        out_type=jax.ShapeDtypeStruct((NC, n, f), jnp.float32),
        scratch_types=[
            pltpu.VMEM((4, 2, 8, CH), jnp.int32),
            pltpu.VMEM((nb, CH, f), jnp.float32),
            pltpu.VMEM_SHARED((n, f), jnp.float32),
            pltpu.SemaphoreType.DMA,
            pltpu.SemaphoreType.DMA,
            pltpu.SemaphoreType.DMA,
            pltpu.SemaphoreType.DMA,
        ],
    )
    def agg_kernel(ei_hbm, mp_hbm, zero_hbm, out_hbm,
                   idx_v, rows_v, agg_sh, isem0, isem1, gsem, ssem):
        c = lax.axis_index("c")
        s = lax.axis_index("s")
        blkbase = (c * NS + s) * nblk
        _tile_row_copy(s, n,
                       lambda o, l: zero_hbm.at[pl.ds(o, l)],
                       lambda o, l: agg_sh.at[pl.ds(o, l)])
        plsc.subcore_barrier()
        _gather_scatter_stream(nblk, nb, ga, ei_hbm, blkbase, mp_hbm,
                               idx_v, rows_v, agg_sh, isem0, isem1,
                               gsem, ssem)
        plsc.subcore_barrier()
        _tile_row_copy(s, n,
                       lambda o, l: agg_sh.at[pl.ds(o, l)],
                       lambda o, l: out_hbm.at[c, pl.ds(o, l)])

    return agg_kernel


@functools.lru_cache(maxsize=None)
def _make_agg_split_kernel(n, e, half, nb, ga):
    nblk = e // NS // (8 * CH)

    @functools.partial(
        pl.kernel,
        mesh=_sc_mesh(),
        compiler_params=pltpu.CompilerParams(use_tc_tiling_on_sc=False),
        out_type=jax.ShapeDtypeStruct((NC * n, half), jnp.float32),
        scratch_types=[
            pltpu.VMEM((4, 2, 8, CH), jnp.int32),
            pltpu.VMEM((nb, CH, half), jnp.float32),
            pltpu.VMEM_SHARED((n, half), jnp.float32),
            pltpu.SemaphoreType.DMA,
            pltpu.SemaphoreType.DMA,
            pltpu.SemaphoreType.DMA,
            pltpu.SemaphoreType.DMA,
        ],
    )
    def agg_kernel(ei_hbm, mplo_hbm, mphi_hbm, zero_hbm, out_hbm,
                   idx_v, rows_v, agg_sh, isem0, isem1, gsem, ssem):
        c = lax.axis_index("c")
        s = lax.axis_index("s")
        blkbase = s * nblk
        _tile_row_copy(s, n,
                       lambda o, l: zero_hbm.at[pl.ds(o, l)],
                       lambda o, l: agg_sh.at[pl.ds(o, l)])
        plsc.subcore_barrier()

        @pl.when(c == 0)
        def _():
            _gather_scatter_stream(nblk, nb, ga, ei_hbm, blkbase, mplo_hbm,
                                   idx_v, rows_v, agg_sh, isem0, isem1,
                                   gsem, ssem)

        @pl.when(c == 1)
        def _():
            _gather_scatter_stream(nblk, nb, ga, ei_hbm, blkbase, mphi_hbm,
                                   idx_v, rows_v, agg_sh, isem0, isem1,
                                   gsem, ssem)

        plsc.subcore_barrier()
        _tile_row_copy(s, n,
                       lambda o, l: agg_sh.at[pl.ds(o, l)],
                       lambda o, l: out_hbm.at[pl.ds(c * n + o, l)])

    return agg_kernel


def _prep_body(degh_ref, x_ref, w_ref, dinv_ref, mplo_ref, mphi_ref):
    half = mplo_ref.shape[1]
    deg = degh_ref[0, :, 0] + degh_ref[1, :, 0] + 1.0
    dinv = lax.rsqrt(deg)[:, None]
    dinv_ref[...] = dinv
    m = dinv * jnp.dot(x_ref[...], w_ref[...])
    mplo_ref[...] = m[:, :half]
    mphi_ref[...] = m[:, half:]


def _mid_split_body(slo_ref, shi_ref, mplo_ref, mphi_ref, dinv_ref, b_ref,
                    w_ref, mn_ref):
    dinv = dinv_ref[...]
    agg = jnp.concatenate([slo_ref[...] + mplo_ref[...],
                           shi_ref[...] + mphi_ref[...]], axis=-1)
    h = jnp.tanh(dinv * agg + b_ref[...])
    mn_ref[...] = dinv * jnp.dot(h, w_ref[...])


def _mid_body(s_ref, mp_ref, dinv_ref, b_ref, w_ref, mn_ref):
    dinv = dinv_ref[...]
    h = jnp.tanh(dinv * (s_ref[0] + s_ref[1] + mp_ref[...]) + b_ref[...])
    mn_ref[...] = dinv * jnp.dot(h, w_ref[...])


def _final_body(s_ref, mp_ref, dinv_ref, b_ref, wc_ref, bc_ref,
                out_ref, h_ref):
    h = jnp.tanh(dinv_ref[...] * (s_ref[0] + s_ref[1] + mp_ref[...])
                 + b_ref[...])
    h_ref[...] = h
    out_ref[...] = jnp.dot(h, wc_ref[...]) + bc_ref[...]


def kernel(x, edge_index, W1, b1, W2, b2, W3, b3, Wc, bc):
    n, d_in = x.shape
    e = edge_index.shape[1]
    f1, f2, f3 = W1.shape[1], W2.shape[1], W3.shape[1]
    half = f1 // 2
    ncls = Wc.shape[1]
    br = 2000
    grid = (n // br,)
    nb = n // br

    ei4 = edge_index.reshape(2, e // (8 * CH), 8, CH)
    ones16 = jnp.ones((CH, 16), jnp.float32)
    zeros = {f: jnp.zeros((n, f), jnp.float32) for f in {16, half, f2, f3}}

    degh = _make_deg_kernel(n, e)(ei4, ones16, zeros[16])

    rows = lambda i: (i, 0)
    rows_hi = lambda i: (i + nb, 0)
    fixed = lambda i: (0, 0)
    part = lambda i: (0, i, 0)

    dinv, mplo, mphi = pl.pallas_call(
        _prep_body,
        grid=grid,
        in_specs=[
            pl.BlockSpec((NC, br, 16), part),
            pl.BlockSpec((br, d_in), rows),
            pl.BlockSpec((d_in, f1), fixed),
        ],
        out_specs=[
            pl.BlockSpec((br, 1), rows),
            pl.BlockSpec((br, half), rows),
            pl.BlockSpec((br, half), rows),
        ],
        out_shape=[
            jax.ShapeDtypeStruct((n, 1), jnp.float32),
            jax.ShapeDtypeStruct((n, half), jnp.float32),
            jax.ShapeDtypeStruct((n, half), jnp.float32),
        ],
    )(degh, x, W1)

    s1f = _make_agg_split_kernel(n, e, half, 9, 6)(
        ei4, mplo, mphi, zeros[half])

    mp2 = pl.pallas_call(
        _mid_split_body,
        grid=grid,
        in_specs=[
            pl.BlockSpec((br, half), rows),
            pl.BlockSpec((br, half), rows_hi),
            pl.BlockSpec((br, half), rows),
            pl.BlockSpec((br, half), rows),
            pl.BlockSpec((br, 1), rows),
            pl.BlockSpec((1, f1), fixed),
            pl.BlockSpec((f1, f2), fixed),
        ],
        out_specs=pl.BlockSpec((br, f2), rows),
        out_shape=jax.ShapeDtypeStruct((n, f2), jnp.float32),
    )(s1f, s1f, mplo, mphi, dinv, b1.reshape(1, f1), W2)

    s2 = _make_agg_kernel(n, e, f2, 9, 6)(ei4, mp2, zeros[f2])

    mp3 = pl.pallas_call(
        _mid_body,
        grid=grid,
        in_specs=[
            pl.BlockSpec((NC, br, f2), part),
            pl.BlockSpec((br, f2), rows),
            pl.BlockSpec((br, 1), rows),
            pl.BlockSpec((1, f2), fixed),
            pl.BlockSpec((f2, f3), fixed),
        ],
        out_specs=pl.BlockSpec((br, f3), rows),
        out_shape=jax.ShapeDtypeStruct((n, f3), jnp.float32),
    )(s2, mp2, dinv, b2.reshape(1, f2), W3)

    s3 = _make_agg_kernel(n, e, f3, 16, 8)(ei4, mp3, zeros[f3])

    out, h3 = pl.pallas_call(
        _final_body,
        grid=grid,
        in_specs=[
            pl.BlockSpec((NC, br, f3), part),
            pl.BlockSpec((br, f3), rows),
            pl.BlockSpec((br, 1), rows),
            pl.BlockSpec((1, f3), fixed),
            pl.BlockSpec((f3, ncls), fixed),
            pl.BlockSpec((1, ncls), fixed),
        ],
        out_specs=[
            pl.BlockSpec((br, ncls), rows),
            pl.BlockSpec((br, f3), rows),
        ],
        out_shape=[
            jax.ShapeDtypeStruct((n, ncls), jnp.float32),
            jax.ShapeDtypeStruct((n, f3), jnp.float32),
        ],
    )(s3, mp3, dinv, b3.reshape(1, f3), Wc, bc.reshape(1, ncls))

    return (out, h3)
